# Initial kernel scaffold; baseline (speedup 1.0000x reference)
import functools

import jax
import jax.numpy as jnp
from jax import lax
from jax.experimental import pallas as pl
from jax.experimental.pallas import tpu as pltpu
from jax.experimental.pallas import tpu_sc as plsc

NC = 2
NS = 16
NW = NC * NS
CB = 80


def _mesh():
    return plsc.VectorSubcoreMesh(core_axis_name="c", subcore_axis_name="s")


@functools.lru_cache(maxsize=None)
def _make_histogram(E, N):
    EPW = E // NW
    NCH = EPW // CB
    assert EPW * NW == E and NCH * CB == EPW

    @functools.partial(
        pl.kernel,
        out_type=jax.ShapeDtypeStruct((NC, N), jnp.float32),
        mesh=_mesh(),
        scratch_types=[
            pltpu.VMEM((NCH, CB), jnp.int32),
            pltpu.VMEM((CB,), jnp.float32),
            pltpu.VMEM_SHARED((N,), jnp.float32),
        ],
    )
    def hist(dst2d_hbm, zeros_hbm, ones_hbm, out_hbm, idx_v, ones_v, acc_sh):
        cid = lax.axis_index("c")
        sid = lax.axis_index("s")
        wid = cid * NS + sid
        pltpu.sync_copy(dst2d_hbm.at[pl.ds(wid * NCH, NCH)], idx_v)
        pltpu.sync_copy(ones_hbm, ones_v)

        @pl.when(sid == 0)
        def _():
            pltpu.sync_copy(zeros_hbm, acc_sh)

        plsc.subcore_barrier()

        def body(j, carry):
            pltpu.sync_copy(ones_v, acc_sh.at[idx_v.at[j]], add=True)
            return carry

        lax.fori_loop(0, NCH, body, 0)
        plsc.subcore_barrier()

        @pl.when(sid == 0)
        def _():
            pltpu.sync_copy(acc_sh, out_hbm.at[cid])

    return hist


@functools.lru_cache(maxsize=None)
def _make_aggregate(E, N, F):
    EPW = E // NW
    NCH = EPW // CB
    assert EPW * NW == E and NCH * CB == EPW
    RPT = N // NS
    assert RPT * NS == N

    @functools.partial(
        pl.kernel,
        out_type=jax.ShapeDtypeStruct((NC, N, F), jnp.float32),
        mesh=_mesh(),
        scratch_types=[
            pltpu.VMEM((NCH, CB), jnp.int32),
            pltpu.VMEM((NCH, CB), jnp.int32),
            pltpu.VMEM((CB, F), jnp.float32),
            pltpu.VMEM_SHARED((N, F), jnp.float32),
            pltpu.SemaphoreType.DMA,
        ],
    )
    def agg(y_hbm, src2d_hbm, dst2d_hbm, zeros_hbm, out_hbm,
            si_v, di_v, rows_v, acc_sh, sem):
        cid = lax.axis_index("c")
        sid = lax.axis_index("s")
        wid = cid * NS + sid
        pltpu.sync_copy(src2d_hbm.at[pl.ds(wid * NCH, NCH)], si_v)
        pltpu.sync_copy(dst2d_hbm.at[pl.ds(wid * NCH, NCH)], di_v)

        @pl.when(sid == 0)
        def _():
            pltpu.sync_copy(zeros_hbm, acc_sh)

        plsc.subcore_barrier()

        def body(j, carry):
            pltpu.async_copy(y_hbm.at[si_v.at[j]], rows_v, sem).wait()
            pltpu.sync_copy(rows_v, acc_sh.at[di_v.at[j]], add=True)
            return carry

        lax.fori_loop(0, NCH, body, 0)
        plsc.subcore_barrier()

        pltpu.sync_copy(acc_sh.at[pl.ds(sid * RPT, RPT)],
                        out_hbm.at[cid, pl.ds(sid * RPT, RPT)])

    return agg


def _tc_stage1(x, W1, c0, c1):
    N, _ = x.shape
    H = W1.shape[1]

    def body(x_ref, w_ref, c0_ref, c1_ref, y_ref, dinv_ref):
        deg = c0_ref[...] + c1_ref[...] + 1.0
        dinv = lax.rsqrt(deg)
        xw = jnp.dot(x_ref[...], w_ref[...], preferred_element_type=jnp.float32)
        y_ref[...] = xw * dinv
        dinv_ref[...] = dinv

    return pl.pallas_call(
        body,
        out_shape=(jax.ShapeDtypeStruct((N, H), jnp.float32),
                   jax.ShapeDtypeStruct((N, 1), jnp.float32)),
    )(x, W1, c0, c1)


def _tc_stage2(z0, z1, y1, dinv, b1, W2):
    N, H = y1.shape
    C = W2.shape[1]

    def body(z0_ref, z1_ref, y1_ref, dinv_ref, b1_ref, w2_ref, y2_ref):
        agg = (z0_ref[...] + z1_ref[...] + y1_ref[...]) * dinv_ref[...] + b1_ref[...]
        h = jnp.maximum(agg, 0.0)
        hw = jnp.dot(h, w2_ref[...], preferred_element_type=jnp.float32)
        y2_ref[...] = hw * dinv_ref[...]

    return pl.pallas_call(
        body,
        out_shape=jax.ShapeDtypeStruct((N, C), jnp.float32),
    )(z0, z1, y1, dinv, b1, W2)


def _tc_stage3(z0, z1, y2, dinv, b2):
    N, C = y2.shape

    def body(z0_ref, z1_ref, y2_ref, dinv_ref, b2_ref, out_ref):
        logits = (z0_ref[...] + z1_ref[...] + y2_ref[...]) * dinv_ref[...] + b2_ref[...]
        m = jnp.max(logits, axis=1, keepdims=True)
        lse = jnp.log(jnp.sum(jnp.exp(logits - m), axis=1, keepdims=True)) + m
        out_ref[...] = logits - lse

    return pl.pallas_call(
        body,
        out_shape=jax.ShapeDtypeStruct((N, C), jnp.float32),
    )(z0, z1, y2, dinv, b2)


def kernel(x, edge_index, W1, b1, W2, b2):
    N, _ = x.shape
    E = edge_index.shape[1]
    H = W1.shape[1]
    C = W2.shape[1]

    src2d = edge_index[0].reshape(E // CB, CB)
    dst2d = edge_index[1].reshape(E // CB, CB)

    zeros_n = jnp.zeros((N,), jnp.float32)
    ones_cb = jnp.ones((CB,), jnp.float32)
    counts = _make_histogram(E, N)(dst2d, zeros_n, ones_cb)
    c0 = counts[0].reshape(N, 1)
    c1 = counts[1].reshape(N, 1)

    y1, dinv = _tc_stage1(x, W1, c0, c1)

    zeros_h = jnp.zeros((N, H), jnp.float32)
    z1 = _make_aggregate(E, N, H)(y1, src2d, dst2d, zeros_h)

    y2 = _tc_stage2(z1[0], z1[1], y1, dinv, b1.reshape(1, H), W2)

    zeros_c = jnp.zeros((N, C), jnp.float32)
    z2 = _make_aggregate(E, N, C)(y2, src2d, dst2d, zeros_c)

    return _tc_stage3(z2[0], z2[1], y2, dinv, b2.reshape(1, C))


# speedup vs baseline: 22.3377x; 22.3377x over previous
import functools

import jax
import jax.numpy as jnp
from jax import lax
from jax.experimental import pallas as pl
from jax.experimental.pallas import tpu as pltpu
from jax.experimental.pallas import tpu_sc as plsc

NC = 2
NS = 16
NW = NC * NS
CB = 80


def _mesh():
    return plsc.VectorSubcoreMesh(core_axis_name="c", subcore_axis_name="s")


_SC_PARAMS = pltpu.CompilerParams(use_tc_tiling_on_sc=False)


@functools.lru_cache(maxsize=None)
def _make_histogram(E, N):
    EPW = E // NW
    NCH = EPW // CB
    assert EPW * NW == E and NCH * CB == EPW

    @functools.partial(
        pl.kernel,
        out_type=jax.ShapeDtypeStruct((NC, N), jnp.float32),
        mesh=_mesh(),
        compiler_params=_SC_PARAMS,
        scratch_types=[
            pltpu.VMEM((NCH, CB), jnp.int32),
            pltpu.VMEM((CB,), jnp.float32),
            pltpu.VMEM_SHARED((N,), jnp.float32),
        ],
    )
    def hist(dst3d_hbm, zeros_hbm, ones_hbm, out_hbm, idx_v, ones_v, acc_sh):
        cid = lax.axis_index("c")
        sid = lax.axis_index("s")
        wid = cid * NS + sid
        pltpu.sync_copy(dst3d_hbm.at[wid], idx_v)
        pltpu.sync_copy(ones_hbm, ones_v)

        @pl.when(sid == 0)
        def _():
            pltpu.sync_copy(zeros_hbm, acc_sh)

        plsc.subcore_barrier()

        def body(j, carry):
            pltpu.sync_copy(ones_v, acc_sh.at[idx_v.at[j]], add=True)
            return carry

        lax.fori_loop(0, NCH, body, 0)
        plsc.subcore_barrier()

        @pl.when(sid == 0)
        def _():
            pltpu.sync_copy(acc_sh, out_hbm.at[cid])

    return hist


@functools.lru_cache(maxsize=None)
def _make_aggregate(E, N, F):
    EPW = E // NW
    NCH = EPW // CB
    assert EPW * NW == E and NCH * CB == EPW
    RPT = N // NS
    assert RPT * NS == N

    @functools.partial(
        pl.kernel,
        out_type=jax.ShapeDtypeStruct((NC, NS, RPT, F), jnp.float32),
        mesh=_mesh(),
        compiler_params=_SC_PARAMS,
        scratch_types=[
            pltpu.VMEM((NCH, CB), jnp.int32),
            pltpu.VMEM((NCH, CB), jnp.int32),
            pltpu.VMEM((CB, F), jnp.float32),
            pltpu.VMEM_SHARED((N, F), jnp.float32),
            pltpu.SemaphoreType.DMA,
        ],
    )
    def agg(y_hbm, src3d_hbm, dst3d_hbm, zeros3d_hbm, out_hbm,
            si_v, di_v, rows_v, acc_sh, sem):
        cid = lax.axis_index("c")
        sid = lax.axis_index("s")
        wid = cid * NS + sid
        pltpu.sync_copy(src3d_hbm.at[wid], si_v)
        pltpu.sync_copy(dst3d_hbm.at[wid], di_v)
        pltpu.sync_copy(zeros3d_hbm.at[sid], acc_sh.at[pl.ds(sid * RPT, RPT)])
        plsc.subcore_barrier()

        def body(j, carry):
            pltpu.async_copy(y_hbm.at[si_v.at[j]], rows_v, sem).wait()
            pltpu.sync_copy(rows_v, acc_sh.at[di_v.at[j]], add=True)
            return carry

        lax.fori_loop(0, NCH, body, 0)
        plsc.subcore_barrier()

        pltpu.sync_copy(acc_sh.at[pl.ds(sid * RPT, RPT)],
                        out_hbm.at[cid, sid])

    return agg


def _tc_stage1(x, W1, c0, c1):
    N, _ = x.shape
    H = W1.shape[1]

    def body(x_ref, w_ref, c0_ref, c1_ref, y_ref, dinv_ref):
        deg = c0_ref[...] + c1_ref[...] + 1.0
        dinv = lax.rsqrt(deg)
        xw = jnp.dot(x_ref[...], w_ref[...], preferred_element_type=jnp.float32)
        y_ref[...] = xw * dinv
        dinv_ref[...] = dinv

    return pl.pallas_call(
        body,
        out_shape=(jax.ShapeDtypeStruct((N, H), jnp.float32),
                   jax.ShapeDtypeStruct((N, 1), jnp.float32)),
    )(x, W1, c0, c1)


def _tc_stage2(z0, z1, y1, dinv, b1, W2):
    N, H = y1.shape
    C = W2.shape[1]

    def body(z0_ref, z1_ref, y1_ref, dinv_ref, b1_ref, w2_ref, y2_ref):
        agg = (z0_ref[...] + z1_ref[...] + y1_ref[...]) * dinv_ref[...] + b1_ref[...]
        h = jnp.maximum(agg, 0.0)
        hw = jnp.dot(h, w2_ref[...], preferred_element_type=jnp.float32)
        y2_ref[...] = hw * dinv_ref[...]

    return pl.pallas_call(
        body,
        out_shape=jax.ShapeDtypeStruct((N, C), jnp.float32),
    )(z0, z1, y1, dinv, b1, W2)


def _tc_stage3(z0, z1, y2, dinv, b2):
    N, C = y2.shape

    def body(z0_ref, z1_ref, y2_ref, dinv_ref, b2_ref, out_ref):
        logits = (z0_ref[...] + z1_ref[...] + y2_ref[...]) * dinv_ref[...] + b2_ref[...]
        m = jnp.max(logits, axis=1, keepdims=True)
        lse = jnp.log(jnp.sum(jnp.exp(logits - m), axis=1, keepdims=True)) + m
        out_ref[...] = logits - lse

    return pl.pallas_call(
        body,
        out_shape=jax.ShapeDtypeStruct((N, C), jnp.float32),
    )(z0, z1, y2, dinv, b2)


def kernel(x, edge_index, W1, b1, W2, b2):
    N, _ = x.shape
    E = edge_index.shape[1]
    H = W1.shape[1]
    C = W2.shape[1]

    src3d = edge_index[0].reshape(NW, E // (NW * CB), CB)
    dst3d = edge_index[1].reshape(NW, E // (NW * CB), CB)

    zeros_n = jnp.zeros((N,), jnp.float32)
    ones_cb = jnp.ones((CB,), jnp.float32)
    counts = _make_histogram(E, N)(dst3d, zeros_n, ones_cb)
    c0 = counts[0].reshape(N, 1)
    c1 = counts[1].reshape(N, 1)

    y1, dinv = _tc_stage1(x, W1, c0, c1)

    zeros_h = jnp.zeros((NS, N // NS, H), jnp.float32)
    z1 = _make_aggregate(E, N, H)(y1, src3d, dst3d, zeros_h).reshape(NC, N, H)

    y2 = _tc_stage2(z1[0], z1[1], y1, dinv, b1.reshape(1, H), W2)

    zeros_c = jnp.zeros((NS, N // NS, C), jnp.float32)
    z2 = _make_aggregate(E, N, C)(y2, src3d, dst3d, zeros_c).reshape(NC, N, C)

    return _tc_stage3(z2[0], z2[1], y2, dinv, b2.reshape(1, C))


# speedup vs baseline: 30.8737x; 1.3821x over previous
import functools

import jax
import jax.numpy as jnp
from jax import lax
from jax.experimental import pallas as pl
from jax.experimental.pallas import tpu as pltpu
from jax.experimental.pallas import tpu_sc as plsc

NC = 2
NS = 16
NW = NC * NS
CB = 80


def _mesh():
    return plsc.VectorSubcoreMesh(core_axis_name="c", subcore_axis_name="s")


_SC_PARAMS = pltpu.CompilerParams(use_tc_tiling_on_sc=False)


@functools.lru_cache(maxsize=None)
def _make_histogram(E, N):
    EPW = E // NW
    NCH = EPW // CB
    assert EPW * NW == E and NCH * CB == EPW

    @functools.partial(
        pl.kernel,
        out_type=jax.ShapeDtypeStruct((NC, N), jnp.float32),
        mesh=_mesh(),
        compiler_params=_SC_PARAMS,
        scratch_types=[
            pltpu.VMEM((NCH, CB), jnp.int32),
            pltpu.VMEM((CB,), jnp.float32),
            pltpu.VMEM_SHARED((N,), jnp.float32),
        ],
    )
    def hist(dst3d_hbm, zeros_hbm, ones_hbm, out_hbm, idx_v, ones_v, acc_sh):
        cid = lax.axis_index("c")
        sid = lax.axis_index("s")
        wid = cid * NS + sid
        pltpu.sync_copy(dst3d_hbm.at[wid], idx_v)
        pltpu.sync_copy(ones_hbm, ones_v)

        @pl.when(sid == 0)
        def _():
            pltpu.sync_copy(zeros_hbm, acc_sh)

        plsc.subcore_barrier()

        def body(j, carry):
            pltpu.sync_copy(ones_v, acc_sh.at[idx_v.at[j]], add=True)
            return carry

        lax.fori_loop(0, NCH, body, 0)
        plsc.subcore_barrier()

        @pl.when(sid == 0)
        def _():
            pltpu.sync_copy(acc_sh, out_hbm.at[cid])

    return hist


@functools.lru_cache(maxsize=None)
def _make_aggregate(E, N, F):
    EPW = E // NW
    NCH = EPW // CB
    assert EPW * NW == E and NCH * CB == EPW
    RPT = N // NS
    assert RPT * NS == N

    assert NCH % 2 == 1 and NCH >= 3
    NPAIR = (NCH - 3) // 2

    @functools.partial(
        pl.kernel,
        out_type=jax.ShapeDtypeStruct((NC, NS, RPT, F), jnp.float32),
        mesh=_mesh(),
        compiler_params=_SC_PARAMS,
        scratch_types=[
            pltpu.VMEM((NCH, CB), jnp.int32),
            pltpu.VMEM((NCH, CB), jnp.int32),
            pltpu.VMEM((CB, F), jnp.float32),
            pltpu.VMEM((CB, F), jnp.float32),
            pltpu.VMEM_SHARED((N, F), jnp.float32),
            pltpu.SemaphoreType.DMA,
            pltpu.SemaphoreType.DMA,
        ],
    )
    def agg(y_hbm, src3d_hbm, dst3d_hbm, zeros3d_hbm, out_hbm,
            si_v, di_v, rows0, rows1, acc_sh, sem0, sem1):
        cid = lax.axis_index("c")
        sid = lax.axis_index("s")
        wid = cid * NS + sid
        pltpu.sync_copy(src3d_hbm.at[wid], si_v)
        pltpu.sync_copy(dst3d_hbm.at[wid], di_v)
        pltpu.sync_copy(zeros3d_hbm.at[sid], acc_sh.at[pl.ds(sid * RPT, RPT)])
        plsc.subcore_barrier()

        pltpu.async_copy(y_hbm.at[si_v.at[0]], rows0, sem0)
        pltpu.async_copy(y_hbm.at[si_v.at[1]], rows1, sem1)

        def body(i, carry):
            g = 2 * i
            pltpu.make_async_copy(y_hbm.at[si_v.at[g]], rows0, sem0).wait()
            pltpu.sync_copy(rows0, acc_sh.at[di_v.at[g]], add=True)
            pltpu.async_copy(y_hbm.at[si_v.at[g + 2]], rows0, sem0)
            pltpu.make_async_copy(y_hbm.at[si_v.at[g + 1]], rows1, sem1).wait()
            pltpu.sync_copy(rows1, acc_sh.at[di_v.at[g + 1]], add=True)
            pltpu.async_copy(y_hbm.at[si_v.at[g + 3]], rows1, sem1)
            return carry

        lax.fori_loop(0, NPAIR, body, 0)

        g0, g1, g2 = NCH - 3, NCH - 2, NCH - 1
        pltpu.make_async_copy(y_hbm.at[si_v.at[g0]], rows0, sem0).wait()
        pltpu.sync_copy(rows0, acc_sh.at[di_v.at[g0]], add=True)
        pltpu.async_copy(y_hbm.at[si_v.at[g2]], rows0, sem0)
        pltpu.make_async_copy(y_hbm.at[si_v.at[g1]], rows1, sem1).wait()
        pltpu.sync_copy(rows1, acc_sh.at[di_v.at[g1]], add=True)
        pltpu.make_async_copy(y_hbm.at[si_v.at[g2]], rows0, sem0).wait()
        pltpu.sync_copy(rows0, acc_sh.at[di_v.at[g2]], add=True)
        plsc.subcore_barrier()

        pltpu.sync_copy(acc_sh.at[pl.ds(sid * RPT, RPT)],
                        out_hbm.at[cid, sid])

    return agg


def _tc_stage1(x, W1, c0, c1):
    N, _ = x.shape
    H = W1.shape[1]

    def body(x_ref, w_ref, c0_ref, c1_ref, y_ref, dinv_ref):
        deg = c0_ref[...] + c1_ref[...] + 1.0
        dinv = lax.rsqrt(deg)
        xw = jnp.dot(x_ref[...], w_ref[...], preferred_element_type=jnp.float32)
        y_ref[...] = xw * dinv
        dinv_ref[...] = dinv

    return pl.pallas_call(
        body,
        out_shape=(jax.ShapeDtypeStruct((N, H), jnp.float32),
                   jax.ShapeDtypeStruct((N, 1), jnp.float32)),
    )(x, W1, c0, c1)


def _tc_stage2(z0, z1, y1, dinv, b1, W2):
    N, H = y1.shape
    C = W2.shape[1]

    def body(z0_ref, z1_ref, y1_ref, dinv_ref, b1_ref, w2_ref, y2_ref):
        agg = (z0_ref[...] + z1_ref[...] + y1_ref[...]) * dinv_ref[...] + b1_ref[...]
        h = jnp.maximum(agg, 0.0)
        hw = jnp.dot(h, w2_ref[...], preferred_element_type=jnp.float32)
        y2_ref[...] = hw * dinv_ref[...]

    return pl.pallas_call(
        body,
        out_shape=jax.ShapeDtypeStruct((N, C), jnp.float32),
    )(z0, z1, y1, dinv, b1, W2)


def _tc_stage3(z0, z1, y2, dinv, b2):
    N, C = y2.shape

    def body(z0_ref, z1_ref, y2_ref, dinv_ref, b2_ref, out_ref):
        logits = (z0_ref[...] + z1_ref[...] + y2_ref[...]) * dinv_ref[...] + b2_ref[...]
        m = jnp.max(logits, axis=1, keepdims=True)
        lse = jnp.log(jnp.sum(jnp.exp(logits - m), axis=1, keepdims=True)) + m
        out_ref[...] = logits - lse

    return pl.pallas_call(
        body,
        out_shape=jax.ShapeDtypeStruct((N, C), jnp.float32),
    )(z0, z1, y2, dinv, b2)


def kernel(x, edge_index, W1, b1, W2, b2):
    N, _ = x.shape
    E = edge_index.shape[1]
    H = W1.shape[1]
    C = W2.shape[1]

    src3d = edge_index[0].reshape(NW, E // (NW * CB), CB)
    dst3d = edge_index[1].reshape(NW, E // (NW * CB), CB)

    zeros_n = jnp.zeros((N,), jnp.float32)
    ones_cb = jnp.ones((CB,), jnp.float32)
    counts = _make_histogram(E, N)(dst3d, zeros_n, ones_cb)
    c0 = counts[0].reshape(N, 1)
    c1 = counts[1].reshape(N, 1)

    y1, dinv = _tc_stage1(x, W1, c0, c1)

    zeros_h = jnp.zeros((NS, N // NS, H), jnp.float32)
    z1 = _make_aggregate(E, N, H)(y1, src3d, dst3d, zeros_h).reshape(NC, N, H)

    y2 = _tc_stage2(z1[0], z1[1], y1, dinv, b1.reshape(1, H), W2)

    zeros_c = jnp.zeros((NS, N // NS, C), jnp.float32)
    z2 = _make_aggregate(E, N, C)(y2, src3d, dst3d, zeros_c).reshape(NC, N, C)

    return _tc_stage3(z2[0], z2[1], y2, dinv, b2.reshape(1, C))


# speedup vs baseline: 35.8757x; 1.1620x over previous
import functools

import jax
import jax.numpy as jnp
from jax import lax
from jax.experimental import pallas as pl
from jax.experimental.pallas import tpu as pltpu
from jax.experimental.pallas import tpu_sc as plsc

NC = 2
NS = 16
NW = NC * NS
CB = 80


def _mesh():
    return plsc.VectorSubcoreMesh(core_axis_name="c", subcore_axis_name="s")


_SC_PARAMS = pltpu.CompilerParams(use_tc_tiling_on_sc=False)


@functools.lru_cache(maxsize=None)
def _make_histogram(E, N):
    EPW = E // NW
    NCH = EPW // CB
    assert EPW * NW == E and NCH * CB == EPW

    @functools.partial(
        pl.kernel,
        out_type=jax.ShapeDtypeStruct((NC, N), jnp.float32),
        mesh=_mesh(),
        compiler_params=_SC_PARAMS,
        scratch_types=[
            pltpu.VMEM((NCH, CB), jnp.int32),
            pltpu.VMEM((CB,), jnp.float32),
            pltpu.VMEM_SHARED((N,), jnp.float32),
        ],
    )
    def hist(edges_hbm, zeros_hbm, ones_hbm, out_hbm, idx_v, ones_v, acc_sh):
        cid = lax.axis_index("c")
        sid = lax.axis_index("s")
        wid = cid * NS + sid
        pltpu.sync_copy(edges_hbm.at[1, wid], idx_v)
        pltpu.sync_copy(ones_hbm, ones_v)

        @pl.when(sid == 0)
        def _():
            pltpu.sync_copy(zeros_hbm, acc_sh)

        plsc.subcore_barrier()

        def body(j, carry):
            pltpu.sync_copy(ones_v, acc_sh.at[idx_v.at[j]], add=True)
            return carry

        lax.fori_loop(0, NCH, body, 0)
        plsc.subcore_barrier()

        @pl.when(sid == 0)
        def _():
            pltpu.sync_copy(acc_sh, out_hbm.at[cid])

    return hist


@functools.lru_cache(maxsize=None)
def _make_aggregate(E, N, F):
    EPW = E // NW
    NCH = EPW // CB
    assert EPW * NW == E and NCH * CB == EPW
    RPT = N // NS
    assert RPT * NS == N
    assert NCH % 2 == 1 and NCH >= 3
    NPAIR = (NCH - 3) // 2

    @functools.partial(
        pl.kernel,
        out_type=jax.ShapeDtypeStruct((NC, NS, RPT, F), jnp.float32),
        mesh=_mesh(),
        compiler_params=_SC_PARAMS,
        scratch_types=[
            pltpu.VMEM((NCH, CB), jnp.int32),
            pltpu.VMEM((NCH, CB), jnp.int32),
            pltpu.VMEM((CB, F), jnp.float32),
            pltpu.VMEM((CB, F), jnp.float32),
            pltpu.VMEM_SHARED((N, F), jnp.float32),
            pltpu.SemaphoreType.DMA,
            pltpu.SemaphoreType.DMA,
        ],
    )
    def agg(y_hbm, edges_hbm, zeros_hbm, out_hbm,
            si_v, di_v, rows0, rows1, acc_sh, sem0, sem1):
        cid = lax.axis_index("c")
        sid = lax.axis_index("s")
        wid = cid * NS + sid
        pltpu.sync_copy(edges_hbm.at[0, wid], si_v)
        pltpu.sync_copy(edges_hbm.at[1, wid], di_v)
        pltpu.sync_copy(zeros_hbm, acc_sh.at[pl.ds(sid * RPT, RPT)])
        plsc.subcore_barrier()

        pltpu.async_copy(y_hbm.at[si_v.at[0]], rows0, sem0)
        pltpu.async_copy(y_hbm.at[si_v.at[1]], rows1, sem1)

        def body(i, carry):
            g = 2 * i
            pltpu.make_async_copy(y_hbm.at[si_v.at[g]], rows0, sem0).wait()
            pltpu.sync_copy(rows0, acc_sh.at[di_v.at[g]], add=True)
            pltpu.async_copy(y_hbm.at[si_v.at[g + 2]], rows0, sem0)
            pltpu.make_async_copy(y_hbm.at[si_v.at[g + 1]], rows1, sem1).wait()
            pltpu.sync_copy(rows1, acc_sh.at[di_v.at[g + 1]], add=True)
            pltpu.async_copy(y_hbm.at[si_v.at[g + 3]], rows1, sem1)
            return carry

        lax.fori_loop(0, NPAIR, body, 0)

        g0, g1, g2 = NCH - 3, NCH - 2, NCH - 1
        pltpu.make_async_copy(y_hbm.at[si_v.at[g0]], rows0, sem0).wait()
        pltpu.sync_copy(rows0, acc_sh.at[di_v.at[g0]], add=True)
        pltpu.async_copy(y_hbm.at[si_v.at[g2]], rows0, sem0)
        pltpu.make_async_copy(y_hbm.at[si_v.at[g1]], rows1, sem1).wait()
        pltpu.sync_copy(rows1, acc_sh.at[di_v.at[g1]], add=True)
        pltpu.make_async_copy(y_hbm.at[si_v.at[g2]], rows0, sem0).wait()
        pltpu.sync_copy(rows0, acc_sh.at[di_v.at[g2]], add=True)
        plsc.subcore_barrier()

        pltpu.sync_copy(acc_sh.at[pl.ds(sid * RPT, RPT)],
                        out_hbm.at[cid, sid])

    return agg


def _tc_stage1(x, W1, counts):
    N, _ = x.shape
    H = W1.shape[1]

    def body(x_ref, w_ref, c_ref, y_ref, dinv_ref):
        deg = c_ref[0] + c_ref[1] + 1.0
        dinv = lax.rsqrt(deg)
        xw = jnp.dot(x_ref[...], w_ref[...], preferred_element_type=jnp.float32)
        y_ref[...] = xw * dinv
        dinv_ref[...] = dinv

    return pl.pallas_call(
        body,
        out_shape=(jax.ShapeDtypeStruct((N, H), jnp.float32),
                   jax.ShapeDtypeStruct((N, 1), jnp.float32)),
    )(x, W1, counts)


def _tc_stage2(z, y1, dinv, b1, W2):
    N, H = y1.shape
    C = W2.shape[1]

    def body(z_ref, y1_ref, dinv_ref, b1_ref, w2_ref, y2_ref):
        agg = (z_ref[0] + z_ref[1] + y1_ref[...]) * dinv_ref[...] + b1_ref[...]
        h = jnp.maximum(agg, 0.0)
        hw = jnp.dot(h, w2_ref[...], preferred_element_type=jnp.float32)
        y2_ref[...] = hw * dinv_ref[...]

    return pl.pallas_call(
        body,
        out_shape=jax.ShapeDtypeStruct((N, C), jnp.float32),
    )(z, y1, dinv, b1, W2)


def _tc_stage3(z, y2, dinv, b2):
    N, C = y2.shape

    def body(z_ref, y2_ref, dinv_ref, b2_ref, out_ref):
        logits = (z_ref[0] + z_ref[1] + y2_ref[...]) * dinv_ref[...] + b2_ref[...]
        m = jnp.max(logits, axis=1, keepdims=True)
        lse = jnp.log(jnp.sum(jnp.exp(logits - m), axis=1, keepdims=True)) + m
        out_ref[...] = logits - lse

    return pl.pallas_call(
        body,
        out_shape=jax.ShapeDtypeStruct((N, C), jnp.float32),
    )(z, y2, dinv, b2)


def kernel(x, edge_index, W1, b1, W2, b2):
    N, _ = x.shape
    E = edge_index.shape[1]
    H = W1.shape[1]
    C = W2.shape[1]
    NCH = E // (NW * CB)
    RPT = N // NS

    edges = edge_index.reshape(2, NW, NCH, CB)

    zeros_n = jnp.zeros((N,), jnp.float32)
    ones_cb = jnp.ones((CB,), jnp.float32)
    counts = _make_histogram(E, N)(edges, zeros_n, ones_cb)

    y1, dinv = _tc_stage1(x, W1, counts.reshape(NC, N, 1))

    zeros_h = jnp.zeros((RPT, H), jnp.float32)
    z1 = _make_aggregate(E, N, H)(y1, edges, zeros_h)

    y2 = _tc_stage2(z1.reshape(NC, N, H), y1, dinv, b1.reshape(1, H), W2)

    zeros_c = jnp.zeros((RPT, C), jnp.float32)
    z2 = _make_aggregate(E, N, C)(y2, edges, zeros_c)

    return _tc_stage3(z2.reshape(NC, N, C), y2, dinv, b2.reshape(1, C))


# speedup vs baseline: 45.8956x; 1.2793x over previous
import functools

import jax
import jax.numpy as jnp
from jax import lax
from jax.experimental import pallas as pl
from jax.experimental.pallas import tpu as pltpu
from jax.experimental.pallas import tpu_sc as plsc

NC = 2
NS = 16
NW = NC * NS
CB = 400


def _mesh():
    return plsc.VectorSubcoreMesh(core_axis_name="c", subcore_axis_name="s")


_SC_PARAMS = pltpu.CompilerParams(use_tc_tiling_on_sc=False)


@functools.lru_cache(maxsize=None)
def _make_histogram(E, N):
    EPW = E // NW
    NCH = EPW // CB
    assert EPW * NW == E and NCH * CB == EPW

    @functools.partial(
        pl.kernel,
        out_type=jax.ShapeDtypeStruct((NC, N), jnp.float32),
        mesh=_mesh(),
        compiler_params=_SC_PARAMS,
        scratch_types=[
            pltpu.VMEM((NCH, CB), jnp.int32),
            pltpu.VMEM((CB,), jnp.float32),
            pltpu.VMEM_SHARED((N,), jnp.float32),
        ],
    )
    def hist(edges_hbm, zeros_hbm, ones_hbm, out_hbm, idx_v, ones_v, acc_sh):
        cid = lax.axis_index("c")
        sid = lax.axis_index("s")
        wid = cid * NS + sid
        pltpu.sync_copy(edges_hbm.at[1, wid], idx_v)
        pltpu.sync_copy(ones_hbm, ones_v)

        @pl.when(sid == 0)
        def _():
            pltpu.sync_copy(zeros_hbm, acc_sh)

        plsc.subcore_barrier()

        def body(j, carry):
            pltpu.sync_copy(ones_v, acc_sh.at[idx_v.at[j]], add=True)
            return carry

        lax.fori_loop(0, NCH, body, 0)
        plsc.subcore_barrier()

        @pl.when(sid == 0)
        def _():
            pltpu.sync_copy(acc_sh, out_hbm.at[cid])

    return hist


@functools.lru_cache(maxsize=None)
def _make_aggregate(E, N, F):
    EPW = E // NW
    NCH = EPW // CB
    assert EPW * NW == E and NCH * CB == EPW
    RPT = N // NS
    assert RPT * NS == N
    assert NCH % 2 == 1 and NCH >= 3
    NPAIR = (NCH - 3) // 2

    @functools.partial(
        pl.kernel,
        out_type=jax.ShapeDtypeStruct((NC, NS, RPT, F), jnp.float32),
        mesh=_mesh(),
        compiler_params=_SC_PARAMS,
        scratch_types=[
            pltpu.VMEM((NCH, CB), jnp.int32),
            pltpu.VMEM((NCH, CB), jnp.int32),
            pltpu.VMEM((CB, F), jnp.float32),
            pltpu.VMEM((CB, F), jnp.float32),
            pltpu.VMEM_SHARED((N, F), jnp.float32),
            pltpu.SemaphoreType.DMA,
            pltpu.SemaphoreType.DMA,
        ],
    )
    def agg(y_hbm, edges_hbm, zeros_hbm, out_hbm,
            si_v, di_v, rows0, rows1, acc_sh, sem0, sem1):
        cid = lax.axis_index("c")
        sid = lax.axis_index("s")
        wid = cid * NS + sid
        pltpu.sync_copy(edges_hbm.at[0, wid], si_v)
        pltpu.sync_copy(edges_hbm.at[1, wid], di_v)
        pltpu.sync_copy(zeros_hbm, acc_sh.at[pl.ds(sid * RPT, RPT)])
        plsc.subcore_barrier()

        pltpu.async_copy(y_hbm.at[si_v.at[0]], rows0, sem0)
        pltpu.async_copy(y_hbm.at[si_v.at[1]], rows1, sem1)

        def body(i, carry):
            g = 2 * i
            pltpu.make_async_copy(y_hbm.at[si_v.at[g]], rows0, sem0).wait()
            pltpu.sync_copy(rows0, acc_sh.at[di_v.at[g]], add=True)
            pltpu.async_copy(y_hbm.at[si_v.at[g + 2]], rows0, sem0)
            pltpu.make_async_copy(y_hbm.at[si_v.at[g + 1]], rows1, sem1).wait()
            pltpu.sync_copy(rows1, acc_sh.at[di_v.at[g + 1]], add=True)
            pltpu.async_copy(y_hbm.at[si_v.at[g + 3]], rows1, sem1)
            return carry

        lax.fori_loop(0, NPAIR, body, 0)

        g0, g1, g2 = NCH - 3, NCH - 2, NCH - 1
        pltpu.make_async_copy(y_hbm.at[si_v.at[g0]], rows0, sem0).wait()
        pltpu.sync_copy(rows0, acc_sh.at[di_v.at[g0]], add=True)
        pltpu.async_copy(y_hbm.at[si_v.at[g2]], rows0, sem0)
        pltpu.make_async_copy(y_hbm.at[si_v.at[g1]], rows1, sem1).wait()
        pltpu.sync_copy(rows1, acc_sh.at[di_v.at[g1]], add=True)
        pltpu.make_async_copy(y_hbm.at[si_v.at[g2]], rows0, sem0).wait()
        pltpu.sync_copy(rows0, acc_sh.at[di_v.at[g2]], add=True)
        plsc.subcore_barrier()

        pltpu.sync_copy(acc_sh.at[pl.ds(sid * RPT, RPT)],
                        out_hbm.at[cid, sid])

    return agg


def _tc_stage1(x, W1, counts):
    N, _ = x.shape
    H = W1.shape[1]

    def body(x_ref, w_ref, c_ref, y_ref, dinv_ref):
        deg = c_ref[0] + c_ref[1] + 1.0
        dinv = lax.rsqrt(deg)
        xw = jnp.dot(x_ref[...], w_ref[...], preferred_element_type=jnp.float32)
        y_ref[...] = xw * dinv
        dinv_ref[...] = dinv

    return pl.pallas_call(
        body,
        out_shape=(jax.ShapeDtypeStruct((N, H), jnp.float32),
                   jax.ShapeDtypeStruct((N, 1), jnp.float32)),
    )(x, W1, counts)


def _tc_stage2(z, y1, dinv, b1, W2):
    N, H = y1.shape
    C = W2.shape[1]

    def body(z_ref, y1_ref, dinv_ref, b1_ref, w2_ref, y2_ref):
        agg = (z_ref[0] + z_ref[1] + y1_ref[...]) * dinv_ref[...] + b1_ref[...]
        h = jnp.maximum(agg, 0.0)
        hw = jnp.dot(h, w2_ref[...], preferred_element_type=jnp.float32)
        y2_ref[...] = hw * dinv_ref[...]

    return pl.pallas_call(
        body,
        out_shape=jax.ShapeDtypeStruct((N, C), jnp.float32),
    )(z, y1, dinv, b1, W2)


def _tc_stage3(z, y2, dinv, b2):
    N, C = y2.shape

    def body(z_ref, y2_ref, dinv_ref, b2_ref, out_ref):
        logits = (z_ref[0] + z_ref[1] + y2_ref[...]) * dinv_ref[...] + b2_ref[...]
        m = jnp.max(logits, axis=1, keepdims=True)
        lse = jnp.log(jnp.sum(jnp.exp(logits - m), axis=1, keepdims=True)) + m
        out_ref[...] = logits - lse

    return pl.pallas_call(
        body,
        out_shape=jax.ShapeDtypeStruct((N, C), jnp.float32),
    )(z, y2, dinv, b2)


def kernel(x, edge_index, W1, b1, W2, b2):
    N, _ = x.shape
    E = edge_index.shape[1]
    H = W1.shape[1]
    C = W2.shape[1]
    NCH = E // (NW * CB)
    RPT = N // NS

    edges = edge_index.reshape(2, NW, NCH, CB)

    zeros_n = jnp.zeros((N,), jnp.float32)
    ones_cb = jnp.ones((CB,), jnp.float32)
    counts = _make_histogram(E, N)(edges, zeros_n, ones_cb)

    y1, dinv = _tc_stage1(x, W1, counts.reshape(NC, N, 1))

    zeros_h = jnp.zeros((RPT, H), jnp.float32)
    z1 = _make_aggregate(E, N, H)(y1, edges, zeros_h)

    y2 = _tc_stage2(z1.reshape(NC, N, H), y1, dinv, b1.reshape(1, H), W2)

    zeros_c = jnp.zeros((RPT, C), jnp.float32)
    z2 = _make_aggregate(E, N, C)(y2, edges, zeros_c)

    return _tc_stage3(z2.reshape(NC, N, C), y2, dinv, b2.reshape(1, C))
